# baseline (speedup 1.0000x reference)
import functools

import jax
import jax.numpy as jnp
from jax import lax
from jax.experimental import pallas as pl
from jax.experimental.pallas import tpu as pltpu

HID = 16
T = 5
START = 3
STOP = 4
NEG = -10000.0


def _bilstm_crf_fused(
    sent_ref,
    emb_hbm, wihf, whhf, bihf, bhhf, wihb, whhb, bihb, bhhb,
    w2t, bt, trans_ref, h0_ref, c0_ref,
    score_ref, path_ref,
    embs, xf_ref, xb_ref, hst_ref, hrev_ref, bpo_ref, sem,
):
    S = path_ref.shape[1]
    f32 = jnp.float32

    copies = [
        pltpu.make_async_copy(
            emb_hbm.at[pl.ds(sent_ref[k], 1)], embs.at[pl.ds(k, 1)], sem)
        for k in range(S)
    ]
    for c in copies:
        c.start()

    z16 = jnp.zeros((HID, HID), f32)
    z112 = jnp.zeros((HID, 112), f32)
    z96 = jnp.zeros((HID, 96), f32)

    def in_cat(w, fwd):
        blocks = []
        for j in range(4):
            blk = w[16 * j:16 * j + 16, :]
            blocks.extend([blk, z16] if fwd else [z16, blk])
        return jnp.concatenate(blocks, axis=0)

    wf_t = in_cat(wihf[...], True).T
    wb_t = in_cat(wihb[...], False).T

    rows = []
    for j in range(4):
        rows.append(jnp.concatenate([whhf[16 * j:16 * j + 16, :], z112], axis=1))
        rows.append(jnp.concatenate([z16, whhb[16 * j:16 * j + 16, :], z96], axis=1))
    whh_t = jnp.concatenate(rows, axis=0).T

    bf = bihf[...] + bhhf[...]
    bb = bihb[...] + bhhb[...]
    bias = jnp.concatenate(
        [x for j in range(4)
         for x in (bf[:, 16 * j:16 * j + 16], bb[:, 16 * j:16 * j + 16])],
        axis=1)

    w2t_t = w2t[...].T
    wa_t = jnp.concatenate([w2t_t[0:16, :], jnp.zeros((112, T), f32)],
                           axis=0)
    wb2_t = jnp.concatenate([jnp.zeros((16, T), f32), w2t_t[16:32, :],
                             jnp.zeros((96, T), f32)], axis=0)

    for c in copies:
        c.wait()
    emb = embs[...]
    xf_ref[...] = jnp.dot(emb, wf_t, preferred_element_type=f32) + bias
    xb_ref[...] = jnp.dot(emb, wb_t, preferred_element_type=f32)

    z1_96 = jnp.zeros((1, 96), f32)
    h = jnp.concatenate([h0_ref[pl.ds(0, 1), :], h0_ref[pl.ds(1, 1), :], z1_96],
                        axis=1)
    c_st = jnp.concatenate([c0_ref[pl.ds(0, 1), :], c0_ref[pl.ds(1, 1), :], z1_96],
                           axis=1)

    for k in range(S):
        kr = S - 1 - k
        x = xf_ref[pl.ds(k, 1), :] + xb_ref[pl.ds(kr, 1), :]
        m = x + jnp.dot(h, whh_t, preferred_element_type=f32)
        sg = jax.nn.sigmoid(m)
        tg = jnp.tanh(m)
        c_st = pltpu.roll(sg, 96, 1) * c_st + sg * pltpu.roll(tg, 64, 1)
        h = pltpu.roll(sg, 32, 1) * jnp.tanh(c_st)
        hst_ref[pl.ds(k, 1), :] = h
        hrev_ref[pl.ds(kr, 1), :] = h

    feats = (jnp.dot(hst_ref[...], wa_t, preferred_element_type=f32)
             + jnp.dot(hrev_ref[...], wb2_t, preferred_element_type=f32)
             + bt[...])
    ft_t = feats.T

    lane_t = lax.broadcasted_iota(jnp.int32, (1, T), 1)
    lane2 = lax.broadcasted_iota(jnp.int32, (T, T), 1)
    sub2 = lax.broadcasted_iota(jnp.int32, (T, T), 0)
    lane_s = lax.broadcasted_iota(jnp.int32, (T, S), 1)
    trans = trans_ref[...]
    trans_t = trans.T

    fv_row = jnp.where(lane_t == START, 0.0, NEG)
    fv_col = None
    bp_cols = jnp.zeros((T, S), jnp.int32)
    for t in range(S):
        if t % 2 == 0:
            nvar = trans + fv_row
            best = jnp.max(nvar, axis=1, keepdims=True)
            bp = jnp.min(jnp.where(nvar == best, lane2, T), axis=1,
                         keepdims=True)
            bp_cols = jnp.where(lane_s == t, bp, bp_cols)
            fv_col = best + ft_t[:, t:t + 1]
        else:
            nvar = trans_t + fv_col
            best = jnp.max(nvar, axis=0, keepdims=True)
            bp = jnp.min(jnp.where(nvar == best, sub2, T), axis=0,
                         keepdims=True)
            bpo_ref[pl.ds(t, 1), :] = bp
            fv_row = best + feats[t:t + 1, :]

    terminal = fv_row + trans_ref[pl.ds(STOP, 1), :]
    path_score = jnp.max(terminal, axis=1, keepdims=True)
    score_ref[...] = path_score
    best_id = jnp.min(jnp.where(terminal == path_score, lane_t, T),
                      axis=1, keepdims=True)

    sub_t = lax.broadcasted_iota(jnp.int32, (T, 1), 0)
    iota_s = lax.broadcasted_iota(jnp.int32, (1, S), 1)
    path_row = jnp.where(iota_s == (S - 1), best_id, 0)
    cur = best_id
    for k in range(S - 1):
        t = S - 1 - k
        if t % 2 == 0:
            bp_t = bp_cols[:, t:t + 1]
            prev = jnp.sum(jnp.where(sub_t == cur, bp_t, 0), axis=0,
                           keepdims=True)
        else:
            bp_t = bpo_ref[pl.ds(t, 1), :]
            prev = jnp.sum(jnp.where(lane_t == cur, bp_t, 0), axis=1,
                           keepdims=True)
        path_row = jnp.where(iota_s == (t - 1), prev, path_row)
        cur = prev
    path_ref[...] = path_row


def kernel(sentence, embedding, w_ih_f, w_hh_f, b_ih_f, b_hh_f,
           w_ih_b, w_hh_b, b_ih_b, b_hh_b, w_h2t, b_h2t, transitions, h0, c0):
    S = sentence.shape[0]
    f32 = jnp.float32

    inputs = (
        embedding,
        w_ih_f, w_hh_f, b_ih_f.reshape(1, 64), b_hh_f.reshape(1, 64),
        w_ih_b, w_hh_b, b_ih_b.reshape(1, 64), b_hh_b.reshape(1, 64),
        w_h2t, b_h2t.reshape(1, T), transitions,
        h0.reshape(2, HID), c0.reshape(2, HID),
    )

    def _vmem_spec(shape):
        nd = len(shape)
        return pl.BlockSpec(shape, lambda *_, _nd=nd: (0,) * _nd)

    in_specs = [pl.BlockSpec(memory_space=pl.ANY)] + [
        _vmem_spec(x.shape) for x in inputs[1:]
    ]

    score, path = pl.pallas_call(
        _bilstm_crf_fused,
        out_shape=(jax.ShapeDtypeStruct((1, 1), f32),
                   jax.ShapeDtypeStruct((1, S), jnp.int32)),
        grid_spec=pltpu.PrefetchScalarGridSpec(
            num_scalar_prefetch=1,
            grid=(1,),
            in_specs=in_specs,
            out_specs=[_vmem_spec((1, 1)), _vmem_spec((1, S))],
            scratch_shapes=[
                pltpu.VMEM((S, HID), f32),
                pltpu.VMEM((S, 128), f32),
                pltpu.VMEM((S, 128), f32),
                pltpu.VMEM((S, 128), f32),
                pltpu.VMEM((S, 128), f32),
                pltpu.VMEM((S, T), jnp.int32),
                pltpu.SemaphoreType.DMA,
            ]),
        compiler_params=pltpu.CompilerParams(
            dimension_semantics=("arbitrary",)),
    )(sentence, *inputs)
    return score[0, 0], path[0, :]


# speedup vs baseline: 1.5721x; 1.5721x over previous
import functools

import jax
import jax.numpy as jnp
from jax import lax
from jax.experimental import pallas as pl
from jax.experimental.pallas import tpu as pltpu

HID = 16
T = 5
START = 3
STOP = 4
NEG = -10000.0


def _bilstm_crf_fused(
    embs, wihf, whhf, bihf, bhhf, wihb, whhb, bihb, bhhb,
    w2t, bt, trans_ref, h0_ref, c0_ref,
    score_ref, path_ref,
    xf_ref, xb_ref, hst_ref, hrev_ref, bpo_ref,
):
    S = path_ref.shape[1]
    f32 = jnp.float32

    z16 = jnp.zeros((HID, HID), f32)
    z112 = jnp.zeros((HID, 112), f32)
    z96 = jnp.zeros((HID, 96), f32)

    def in_cat(w, fwd):
        blocks = []
        for j in range(4):
            blk = w[16 * j:16 * j + 16, :]
            blocks.extend([blk, z16] if fwd else [z16, blk])
        return jnp.concatenate(blocks, axis=0)

    wf_t = in_cat(wihf[...], True).T
    wb_t = in_cat(wihb[...], False).T

    rows = []
    for j in range(4):
        rows.append(jnp.concatenate([whhf[16 * j:16 * j + 16, :], z112], axis=1))
        rows.append(jnp.concatenate([z16, whhb[16 * j:16 * j + 16, :], z96], axis=1))
    whh_t = jnp.concatenate(rows, axis=0).T

    bf = bihf[...] + bhhf[...]
    bb = bihb[...] + bhhb[...]
    bias = jnp.concatenate(
        [x for j in range(4)
         for x in (bf[:, 16 * j:16 * j + 16], bb[:, 16 * j:16 * j + 16])],
        axis=1)

    w2t_t = w2t[...].T
    wa_t = jnp.concatenate([w2t_t[0:16, :], jnp.zeros((112, T), f32)],
                           axis=0)
    wb2_t = jnp.concatenate([jnp.zeros((16, T), f32), w2t_t[16:32, :],
                             jnp.zeros((96, T), f32)], axis=0)

    emb = embs[...]
    xf_ref[...] = jnp.dot(emb, wf_t, preferred_element_type=f32) + bias
    xb_ref[...] = jnp.dot(emb, wb_t, preferred_element_type=f32)

    z1_96 = jnp.zeros((1, 96), f32)
    h = jnp.concatenate([h0_ref[pl.ds(0, 1), :], h0_ref[pl.ds(1, 1), :], z1_96],
                        axis=1)
    c_st = jnp.concatenate([c0_ref[pl.ds(0, 1), :], c0_ref[pl.ds(1, 1), :], z1_96],
                           axis=1)

    for k in range(S):
        kr = S - 1 - k
        x = xf_ref[pl.ds(k, 1), :] + xb_ref[pl.ds(kr, 1), :]
        m = x + jnp.dot(h, whh_t, preferred_element_type=f32)
        sg = jax.nn.sigmoid(m)
        tg = jnp.tanh(m)
        c_st = pltpu.roll(sg, 96, 1) * c_st + sg * pltpu.roll(tg, 64, 1)
        h = pltpu.roll(sg, 32, 1) * jnp.tanh(c_st)
        hst_ref[pl.ds(k, 1), :] = h
        hrev_ref[pl.ds(kr, 1), :] = h

    feats = (jnp.dot(hst_ref[...], wa_t, preferred_element_type=f32)
             + jnp.dot(hrev_ref[...], wb2_t, preferred_element_type=f32)
             + bt[...])
    ft_t = feats.T

    lane_t = lax.broadcasted_iota(jnp.int32, (1, T), 1)
    lane2 = lax.broadcasted_iota(jnp.int32, (T, T), 1)
    sub2 = lax.broadcasted_iota(jnp.int32, (T, T), 0)
    lane_s = lax.broadcasted_iota(jnp.int32, (T, S), 1)
    trans = trans_ref[...]
    trans_t = trans.T

    fv_row = jnp.where(lane_t == START, 0.0, NEG)
    fv_col = None
    bp_cols = jnp.zeros((T, S), jnp.int32)
    for t in range(S):
        if t % 2 == 0:
            nvar = trans + fv_row
            best = jnp.max(nvar, axis=1, keepdims=True)
            bp = jnp.min(jnp.where(nvar == best, lane2, T), axis=1,
                         keepdims=True)
            bp_cols = jnp.where(lane_s == t, bp, bp_cols)
            fv_col = best + ft_t[:, t:t + 1]
        else:
            nvar = trans_t + fv_col
            best = jnp.max(nvar, axis=0, keepdims=True)
            bp = jnp.min(jnp.where(nvar == best, sub2, T), axis=0,
                         keepdims=True)
            bpo_ref[pl.ds(t, 1), :] = bp
            fv_row = best + feats[t:t + 1, :]

    terminal = fv_row + trans_ref[pl.ds(STOP, 1), :]
    path_score = jnp.max(terminal, axis=1, keepdims=True)
    score_ref[...] = path_score
    best_id = jnp.min(jnp.where(terminal == path_score, lane_t, T),
                      axis=1, keepdims=True)

    sub_t = lax.broadcasted_iota(jnp.int32, (T, 1), 0)
    iota_s = lax.broadcasted_iota(jnp.int32, (1, S), 1)
    path_row = jnp.where(iota_s == (S - 1), best_id, 0)
    cur = best_id
    for k in range(S - 1):
        t = S - 1 - k
        if t % 2 == 0:
            bp_t = bp_cols[:, t:t + 1]
            prev = jnp.sum(jnp.where(sub_t == cur, bp_t, 0), axis=0,
                           keepdims=True)
        else:
            bp_t = bpo_ref[pl.ds(t, 1), :]
            prev = jnp.sum(jnp.where(lane_t == cur, bp_t, 0), axis=1,
                           keepdims=True)
        path_row = jnp.where(iota_s == (t - 1), prev, path_row)
        cur = prev
    path_ref[...] = path_row


def kernel(sentence, embedding, w_ih_f, w_hh_f, b_ih_f, b_hh_f,
           w_ih_b, w_hh_b, b_ih_b, b_hh_b, w_h2t, b_h2t, transitions, h0, c0):
    S = sentence.shape[0]
    f32 = jnp.float32

    inputs = (
        embedding[sentence],
        w_ih_f, w_hh_f, b_ih_f.reshape(1, 64), b_hh_f.reshape(1, 64),
        w_ih_b, w_hh_b, b_ih_b.reshape(1, 64), b_hh_b.reshape(1, 64),
        w_h2t, b_h2t.reshape(1, T), transitions,
        h0.reshape(2, HID), c0.reshape(2, HID),
    )

    def _vmem_spec(shape):
        nd = len(shape)
        return pl.BlockSpec(shape, lambda *_, _nd=nd: (0,) * _nd)

    in_specs = [_vmem_spec(x.shape) for x in inputs]

    score, path = pl.pallas_call(
        _bilstm_crf_fused,
        out_shape=(jax.ShapeDtypeStruct((1, 1), f32),
                   jax.ShapeDtypeStruct((1, S), jnp.int32)),
        grid_spec=pltpu.PrefetchScalarGridSpec(
            num_scalar_prefetch=0,
            grid=(1,),
            in_specs=in_specs,
            out_specs=[_vmem_spec((1, 1)), _vmem_spec((1, S))],
            scratch_shapes=[
                pltpu.VMEM((S, 128), f32),
                pltpu.VMEM((S, 128), f32),
                pltpu.VMEM((S, 128), f32),
                pltpu.VMEM((S, 128), f32),
                pltpu.VMEM((S, T), jnp.int32),
            ]),
        compiler_params=pltpu.CompilerParams(
            dimension_semantics=("arbitrary",)),
    )(*inputs)
    return score[0, 0], path[0, :]
